# Initial kernel scaffold; baseline (speedup 1.0000x reference)
import jax
import jax.numpy as jnp
from jax.experimental import pallas as pl


def kernel(boxes, scores):
    raise NotImplementedError("write your pallas kernel here")


# speedup vs baseline: 21.3571x; 21.3571x over previous
import jax
import jax.numpy as jnp
from jax import lax
from jax.experimental import pallas as pl
from jax.experimental.pallas import tpu as pltpu

_N = 20000
_NPAD = 20480
_ROWS = 160
_MAX_DET = 300
_IOU_THRESH = 0.5
_SCORE_THRESH = 0.05
_NEG = -1e30


def _nms_body(coords_ref, scores_ref, out_ref):
    x1 = coords_ref[0]
    y1 = coords_ref[1]
    x2 = coords_ref[2]
    y2 = coords_ref[3]
    s = scores_ref[...]

    work0 = jnp.where(s > _SCORE_THRESH, s, _NEG)
    area = (x2 - x1) * (y2 - y1)
    ridx = lax.broadcasted_iota(jnp.int32, (_ROWS, 128), 0)
    cidx = lax.broadcasted_iota(jnp.int32, (_ROWS, 128), 1)
    idxs = ridx * 128 + cidx
    lane = lax.broadcasted_iota(jnp.int32, (1, 128), 1)

    def step(t, work):
        m = jnp.max(work)
        selm = work == m
        idx = jnp.min(jnp.where(selm, idxs, jnp.int32(2**31 - 1)))
        issel = idxs == idx
        bx1 = jnp.max(jnp.where(issel, x1, _NEG))
        by1 = jnp.max(jnp.where(issel, y1, _NEG))
        bx2 = jnp.max(jnp.where(issel, x2, _NEG))
        by2 = jnp.max(jnp.where(issel, y2, _NEG))
        ix1 = jnp.maximum(bx1, x1)
        iy1 = jnp.maximum(by1, y1)
        ix2 = jnp.minimum(bx2, x2)
        iy2 = jnp.minimum(by2, y2)
        inter = jnp.maximum(ix2 - ix1, 0.0) * jnp.maximum(iy2 - iy1, 0.0)
        a1 = (bx2 - bx1) * (by2 - by1)
        iou = inter / (a1 + area - inter + 1e-6)
        sup = iou > _IOU_THRESH
        nw = jnp.where(sup | issel, _NEG, work)
        valid = m > _SCORE_THRESH
        vf = jnp.where(valid, 1.0, 0.0)
        sc = jnp.where(valid, m, 0.0)
        row = jnp.where(
            lane == 0, bx1 * vf,
            jnp.where(lane == 1, by1 * vf,
                      jnp.where(lane == 2, bx2 * vf,
                                jnp.where(lane == 3, by2 * vf,
                                          jnp.where(lane == 4, sc, 0.0)))))
        out_ref[pl.ds(t, 1), :] = row
        return nw

    lax.fori_loop(0, _MAX_DET, step, work0)


def kernel(boxes, scores):
    x1 = boxes[:, 0] * 800.0
    y1 = boxes[:, 1] * 800.0
    x2 = x1 + boxes[:, 2] * 96.0 + 1.0
    y2 = y1 + boxes[:, 3] * 96.0 + 1.0
    coords = jnp.stack([x1, y1, x2, y2], axis=0)
    coords = jnp.pad(coords, ((0, 0), (0, _NPAD - _N)))
    coords = coords.reshape(4, _ROWS, 128)
    s = jnp.pad(scores, (0, _NPAD - _N))
    s = s.reshape(_ROWS, 128)

    out = pl.pallas_call(
        _nms_body,
        out_shape=jax.ShapeDtypeStruct((_MAX_DET, 128), jnp.float32),
        in_specs=[
            pl.BlockSpec(memory_space=pltpu.VMEM),
            pl.BlockSpec(memory_space=pltpu.VMEM),
        ],
        out_specs=pl.BlockSpec(memory_space=pltpu.VMEM),
    )(coords, s)
    return out[:, :5]
